# baseline (speedup 1.0000x reference)
import functools

import jax
import jax.numpy as jnp
from jax import lax
from jax.experimental import pallas as pl
from jax.experimental.pallas import tpu as pltpu
from jax.experimental.pallas import tpu_sc as plsc

NBINS = 8192
DIM = 256
TM = 1024

NC = 2
NS = 16
NW = NC * NS
CH = 128


def _argmax_body(h_ref, h2_ref, et2_ref, e2_ref, out_ref):
    mm2 = lax.dot_general(
        h_ref[...], et2_ref[...], (((1,), (1,)), ((), ())),
        preferred_element_type=jnp.float32,
    )
    t = (h2_ref[...] - mm2) + e2_ref[...]
    out_ref[...] = jnp.argmin(t, axis=1).astype(jnp.int32)


def _encode_indices(h, h2, embed_t, e2):
    ntok = h.shape[0]
    return pl.pallas_call(
        _argmax_body,
        grid=(ntok // TM,),
        in_specs=[
            pl.BlockSpec((TM, DIM), lambda i: (i, 0)),
            pl.BlockSpec((TM, 1), lambda i: (i, 0)),
            pl.BlockSpec((NBINS, DIM), lambda i: (0, 0)),
            pl.BlockSpec((1, NBINS), lambda i: (0, 0)),
        ],
        out_specs=pl.BlockSpec((TM,), lambda i: (i,)),
        out_shape=jax.ShapeDtypeStruct((ntok,), jnp.int32),
    )(h, h2, embed_t, e2)


def _make_sc_gather(ntok):
    bpw = ntok // NW
    nch = bpw // CH
    mesh = plsc.VectorSubcoreMesh(core_axis_name="c", subcore_axis_name="s")

    nbuf = 3

    @functools.partial(
        pl.kernel,
        mesh=mesh,
        out_type=jax.ShapeDtypeStruct((ntok, DIM), jnp.float32),
        scratch_types=[
            pltpu.VMEM((nch, CH), jnp.int32),
        ] + [pltpu.VMEM((CH, DIM), jnp.float32) for _ in range(nbuf)]
          + [pltpu.SemaphoreType.DMA for _ in range(2 * nbuf)],
    )
    def gather_kernel(table_hbm, idx_hbm, out_hbm, idx_v, *bufs_sems):
        bufs = bufs_sems[:nbuf]
        gsems = bufs_sems[nbuf:2 * nbuf]
        wsems = bufs_sems[2 * nbuf:]
        wid = lax.axis_index("s") * NC + lax.axis_index("c")
        pltpu.sync_copy(idx_hbm.at[pl.ds(wid * nch, nch)], idx_v)
        gcp = [None] * nch
        wcp = [None] * nbuf
        for c in range(min(2, nch)):
            gcp[c] = pltpu.async_copy(table_hbm.at[idx_v.at[c]],
                                      bufs[c % nbuf], gsems[c % nbuf])
        for c in range(nch):
            b = c % nbuf
            gcp[c].wait()
            wcp[b] = pltpu.async_copy(
                bufs[b], out_hbm.at[pl.ds(wid * bpw + c * CH, CH)], wsems[b])
            n = c + 2
            if n < nch:
                nb = n % nbuf
                if wcp[nb] is not None:
                    wcp[nb].wait()
                    wcp[nb] = None
                gcp[n] = pltpu.async_copy(table_hbm.at[idx_v.at[n]],
                                          bufs[nb], gsems[nb])
        for b in range(nbuf):
            if wcp[b] is not None:
                wcp[b].wait()

    return gather_kernel


def kernel(hidden_states, embed):
    shape = hidden_states.shape
    h = hidden_states.reshape((-1, shape[-1]))
    ntok = h.shape[0]
    embed_t = embed.T
    h2 = jnp.sum(h ** 2, axis=1, keepdims=True)
    e2 = jnp.sum(embed_t ** 2, axis=0, keepdims=True)
    idx = _encode_indices(h, h2, embed + embed, e2)
    idx2 = idx.reshape((ntok // CH, CH))
    quantize = _make_sc_gather(ntok)(embed, idx2)
    return quantize.reshape(shape)


# speedup vs baseline: 1.0346x; 1.0346x over previous
import functools

import jax
import jax.numpy as jnp
from jax import lax
from jax.experimental import pallas as pl
from jax.experimental.pallas import tpu as pltpu
from jax.experimental.pallas import tpu_sc as plsc

NBINS = 8192
DIM = 256
TM = 512

NC = 2
NS = 16
NW = NC * NS
CH = 128


def _argmax_body(h_ref, h2_ref, et2_ref, e2_ref, out_ref):
    mm2 = lax.dot_general(
        h_ref[...], et2_ref[...], (((1,), (1,)), ((), ())),
        preferred_element_type=jnp.float32,
    )
    t = (h2_ref[...] - mm2) + e2_ref[...]
    out_ref[...] = jnp.argmin(t, axis=1).astype(jnp.int32)


def _encode_indices(h, h2, embed_t, e2):
    ntok = h.shape[0]
    return pl.pallas_call(
        _argmax_body,
        grid=(ntok // TM,),
        in_specs=[
            pl.BlockSpec((TM, DIM), lambda i: (i, 0)),
            pl.BlockSpec((TM, 1), lambda i: (i, 0)),
            pl.BlockSpec((NBINS, DIM), lambda i: (0, 0)),
            pl.BlockSpec((1, NBINS), lambda i: (0, 0)),
        ],
        out_specs=pl.BlockSpec((TM,), lambda i: (i,)),
        out_shape=jax.ShapeDtypeStruct((ntok,), jnp.int32),
    )(h, h2, embed_t, e2)


def _make_sc_gather(ntok):
    bpw = ntok // NW
    nch = bpw // CH
    mesh = plsc.VectorSubcoreMesh(core_axis_name="c", subcore_axis_name="s")

    nbuf = 3

    @functools.partial(
        pl.kernel,
        mesh=mesh,
        out_type=jax.ShapeDtypeStruct((ntok, DIM), jnp.float32),
        scratch_types=[
            pltpu.VMEM((nch, CH), jnp.int32),
        ] + [pltpu.VMEM((CH, DIM), jnp.float32) for _ in range(nbuf)]
          + [pltpu.SemaphoreType.DMA for _ in range(2 * nbuf)],
    )
    def gather_kernel(table_hbm, idx_hbm, out_hbm, idx_v, *bufs_sems):
        bufs = bufs_sems[:nbuf]
        gsems = bufs_sems[nbuf:2 * nbuf]
        wsems = bufs_sems[2 * nbuf:]
        wid = lax.axis_index("s") * NC + lax.axis_index("c")
        pltpu.sync_copy(idx_hbm.at[pl.ds(wid * nch, nch)], idx_v)
        gcp = [None] * nch
        wcp = [None] * nbuf
        for c in range(min(2, nch)):
            gcp[c] = pltpu.async_copy(table_hbm.at[idx_v.at[c]],
                                      bufs[c % nbuf], gsems[c % nbuf])
        for c in range(nch):
            b = c % nbuf
            gcp[c].wait()
            wcp[b] = pltpu.async_copy(
                bufs[b], out_hbm.at[pl.ds(wid * bpw + c * CH, CH)], wsems[b])
            n = c + 2
            if n < nch:
                nb = n % nbuf
                if wcp[nb] is not None:
                    wcp[nb].wait()
                    wcp[nb] = None
                gcp[n] = pltpu.async_copy(table_hbm.at[idx_v.at[n]],
                                          bufs[nb], gsems[nb])
        for b in range(nbuf):
            if wcp[b] is not None:
                wcp[b].wait()

    return gather_kernel


def kernel(hidden_states, embed):
    shape = hidden_states.shape
    h = hidden_states.reshape((-1, shape[-1]))
    ntok = h.shape[0]
    embed_t = embed.T
    h2 = jnp.sum(h ** 2, axis=1, keepdims=True)
    e2 = jnp.sum(embed_t ** 2, axis=0, keepdims=True)
    idx = _encode_indices(h, h2, embed + embed, e2)
    idx2 = idx.reshape((ntok // CH, CH))
    quantize = _make_sc_gather(ntok)(embed, idx2)
    return quantize.reshape(shape)
